# baseline (speedup 1.0000x reference)
import functools

import jax
import jax.numpy as jnp
from jax import lax
from jax.experimental import pallas as pl
from jax.experimental.pallas import tpu as pltpu
from jax.experimental.pallas import tpu_sc as plsc

_DIM = 50
_DIMP = 56
_BATCH = 16384
_NC = 2
_NS = 16
_L = 16
_NW = _NC * _NS
_BPW = _BATCH // _NW
_CH = 128
_NCH = _BPW // _CH

_mesh = plsc.VectorSubcoreMesh(core_axis_name="c", subcore_axis_name="s")


@functools.partial(
    pl.kernel,
    out_type=jax.ShapeDtypeStruct((_BATCH,), jnp.float32),
    mesh=_mesh,
    scratch_types=[
        pltpu.VMEM((_BPW,), jnp.int32),
        pltpu.VMEM((_BPW,), jnp.int32),
        pltpu.VMEM((_BPW, _DIMP), jnp.float32),
        pltpu.VMEM((_BPW, _DIMP), jnp.float32),
        pltpu.VMEM((_BPW,), jnp.float32),
        pltpu.SemaphoreType.DMA((_NCH,)),
        pltpu.SemaphoreType.DMA((_NCH,)),
    ],
    compiler_params=pltpu.CompilerParams(
        use_tc_tiling_on_sc=False, needs_layout_passes=False),
)
def _sc_dot(iw_hbm, ic_hbm, words_hbm, ctx_hbm, out_hbm,
            iw_v, ic_v, wrows, crows, outv, wsem, csem):
    wid = lax.axis_index("s") * _NC + lax.axis_index("c")
    base = wid * _BPW

    pltpu.sync_copy(iw_hbm.at[pl.ds(base, _BPW)], iw_v)
    pltpu.sync_copy(ic_hbm.at[pl.ds(base, _BPW)], ic_v)

    wcp, ccp = [], []
    for j in range(_NCH):
        s = pl.ds(j * _CH, _CH)
        wcp.append(pltpu.async_copy(words_hbm.at[iw_v.at[s]], wrows.at[s],
                                    wsem.at[j]))
        ccp.append(pltpu.async_copy(ctx_hbm.at[ic_v.at[s]], crows.at[s],
                                    csem.at[j]))

    for j in range(_NCH):
        wcp[j].wait()
        ccp[j].wait()

        def group(g, carry, j=j):
            rows = j * _CH + g * _L + lax.iota(jnp.int32, _L)
            acc = jnp.zeros((_L,), jnp.float32)
            for d in range(_DIM):
                col = jnp.full((_L,), d, jnp.int32)
                w = plsc.load_gather(wrows, [rows, col])
                c = plsc.load_gather(crows, [rows, col])
                acc = acc + w * c
            outv[pl.ds(j * _CH + g * _L, _L)] = acc
            return carry

        lax.fori_loop(0, _CH // _L, group, 0)

    pltpu.sync_copy(outv, out_hbm.at[pl.ds(base, _BPW)])


def kernel(i_w, i_c, words, contexts):
    wp = jnp.pad(words, ((0, 0), (0, _DIMP - _DIM)))
    cp = jnp.pad(contexts, ((0, 0), (0, _DIMP - _DIM)))
    out = _sc_dot(i_w.astype(jnp.int32), i_c.astype(jnp.int32), wp, cp)
    return out.reshape(_BATCH, 1, 1)


# speedup vs baseline: 1.0074x; 1.0074x over previous
import functools

import jax
import jax.numpy as jnp
from jax import lax
from jax.experimental import pallas as pl
from jax.experimental.pallas import tpu as pltpu
from jax.experimental.pallas import tpu_sc as plsc

_DIM = 50
_DIMP = 56
_BATCH = 16384
_NC = 2
_NS = 16
_L = 16
_NW = _NC * _NS
_BPW = _BATCH // _NW
_CH = 128
_NCH = _BPW // _CH

_mesh = plsc.VectorSubcoreMesh(core_axis_name="c", subcore_axis_name="s")


@functools.partial(
    pl.kernel,
    out_type=jax.ShapeDtypeStruct((_BATCH,), jnp.float32),
    mesh=_mesh,
    scratch_types=[
        pltpu.VMEM((_BPW,), jnp.int32),
        pltpu.VMEM((_BPW,), jnp.int32),
        pltpu.VMEM((_BPW, _DIMP), jnp.float32),
        pltpu.VMEM((_BPW, _DIMP), jnp.float32),
        pltpu.VMEM((_BPW,), jnp.float32),
        pltpu.SemaphoreType.DMA,
    ],
    compiler_params=pltpu.CompilerParams(
        use_tc_tiling_on_sc=False, needs_layout_passes=False),
)
def _sc_dot(iw_hbm, ic_hbm, words_hbm, ctx_hbm, out_hbm,
            iw_v, ic_v, wrows, crows, outv, sem):
    wid = lax.axis_index("s") * _NC + lax.axis_index("c")
    base = wid * _BPW

    pltpu.sync_copy(iw_hbm.at[pl.ds(base, _BPW)], iw_v)
    pltpu.sync_copy(ic_hbm.at[pl.ds(base, _BPW)], ic_v)

    copies = []
    for j in range(_NCH):
        s = pl.ds(j * _CH, _CH)
        copies.append(pltpu.async_copy(words_hbm.at[iw_v.at[s]], wrows.at[s], sem))
        copies.append(pltpu.async_copy(ctx_hbm.at[ic_v.at[s]], crows.at[s], sem))
    for cp in copies:
        cp.wait()

    def group(g, carry):
        rows = g * _L + lax.iota(jnp.int32, _L)
        acc = jnp.zeros((_L,), jnp.float32)
        for d in range(_DIM):
            col = jnp.full((_L,), d, jnp.int32)
            w = plsc.load_gather(wrows, [rows, col])
            c = plsc.load_gather(crows, [rows, col])
            acc = acc + w * c
        outv[pl.ds(g * _L, _L)] = acc
        return carry

    lax.fori_loop(0, _BPW // _L, group, 0)

    pltpu.sync_copy(outv, out_hbm.at[pl.ds(base, _BPW)])


def kernel(i_w, i_c, words, contexts):
    wp = jnp.pad(words, ((0, 0), (0, _DIMP - _DIM)))
    cp = jnp.pad(contexts, ((0, 0), (0, _DIMP - _DIM)))
    out = _sc_dot(i_w.astype(jnp.int32), i_c.astype(jnp.int32), wp, cp)
    return out.reshape(_BATCH, 1, 1)
